# baseline (speedup 1.0000x reference)
import functools

import jax
import jax.numpy as jnp
from jax import lax
from jax.experimental import pallas as pl
from jax.experimental.pallas import tpu as pltpu
from jax.experimental.pallas import tpu_sc as plsc

ENT_NUM = 100000
DIM = 64
B = 16384
GAMMA = 12.0

_NC = 2
_NS = 16
_NW = _NC * _NS
_BPW = B // _NW
_CHUNK = 256
_NCHUNK = _BPW // _CHUNK


def _concat_body(a_ref, b_ref, o_ref):
    ab = jnp.concatenate([a_ref[...], b_ref[...]], axis=0)
    eye = jnp.eye(2 * DIM, dtype=jnp.float32)
    o_ref[...] = jax.lax.dot_general(
        ab, eye, (((0,), (0,)), ((), ())),
        preferred_element_type=jnp.float32)


def _tc_concat(a, b):
    blk = 1024
    spec = pl.BlockSpec((DIM, blk), lambda i: (0, i))
    return pl.pallas_call(
        _concat_body,
        grid=(pl.cdiv(ENT_NUM, blk),),
        in_specs=[spec, spec],
        out_specs=pl.BlockSpec((blk, 2 * DIM), lambda i: (i, 0)),
        out_shape=jax.ShapeDtypeStruct((ENT_NUM, 2 * DIM), jnp.float32),
    )(a, b)


def _sc_gather2(table, idx0, idx1):
    mesh = plsc.VectorSubcoreMesh(core_axis_name="c", subcore_axis_name="s")
    out_t = [jax.ShapeDtypeStruct((B, 2 * DIM), jnp.float32) for _ in range(2)]

    @functools.partial(
        pl.kernel,
        mesh=mesh,
        out_type=out_t,
        scratch_types=[
            pltpu.VMEM((_BPW,), jnp.int32),
            pltpu.VMEM((_BPW,), jnp.int32),
            pltpu.VMEM((_CHUNK, 2 * DIM), jnp.float32),
            pltpu.VMEM((_CHUNK, 2 * DIM), jnp.float32),
            pltpu.SemaphoreType.DMA,
        ],
    )
    def k(t_hbm, i0_hbm, i1_hbm, o0, o1, i0_v, i1_v, b0, b1, sem):
        wid = lax.axis_index("s") * _NC + lax.axis_index("c")
        base = wid * _BPW
        pltpu.sync_copy(i0_hbm.at[pl.ds(base, _BPW)], i0_v)
        pltpu.sync_copy(i1_hbm.at[pl.ds(base, _BPW)], i1_v)

        def body(c, _):
            off = c * _CHUNK
            cp0 = pltpu.async_copy(t_hbm.at[i0_v.at[pl.ds(off, _CHUNK)]], b0, sem)
            cp1 = pltpu.async_copy(t_hbm.at[i1_v.at[pl.ds(off, _CHUNK)]], b1, sem)
            cp0.wait()
            cp1.wait()
            dst = pl.ds(base + off, _CHUNK)
            pltpu.sync_copy(b0, o0.at[dst])
            pltpu.sync_copy(b1, o1.at[dst])

        lax.fori_loop(0, _NCHUNK, body, None, unroll=False)

    return k(table, idx0, idx1)


def _sc_gather1(table, idx0):
    mesh = plsc.VectorSubcoreMesh(core_axis_name="c", subcore_axis_name="s")
    out_t = jax.ShapeDtypeStruct((B, 2 * DIM), jnp.float32)

    @functools.partial(
        pl.kernel,
        mesh=mesh,
        out_type=out_t,
        scratch_types=[
            pltpu.VMEM((_BPW,), jnp.int32),
            pltpu.VMEM((_CHUNK, 2 * DIM), jnp.float32),
            pltpu.SemaphoreType.DMA,
        ],
    )
    def k(t_hbm, i0_hbm, o0, i0_v, b0, sem):
        wid = lax.axis_index("s") * _NC + lax.axis_index("c")
        base = wid * _BPW
        pltpu.sync_copy(i0_hbm.at[pl.ds(base, _BPW)], i0_v)

        def body(c, _):
            off = c * _CHUNK
            cp0 = pltpu.async_copy(t_hbm.at[i0_v.at[pl.ds(off, _CHUNK)]], b0, sem)
            cp0.wait()
            pltpu.sync_copy(b0, o0.at[pl.ds(base + off, _CHUNK)])

        lax.fori_loop(0, _NCHUNK, body, None, unroll=False)

    return k(table, idx0)


def _tc_body(h_ref, t_ref, r_ref, o_ref):
    h = h_ref[...]
    t = t_ref[...]
    rw = r_ref[...]
    h_re = h[:, :DIM]
    h_im = h[:, DIM:]
    t_re = t[:, :DIM]
    t_im = t[:, DIM:]
    r = rw[:, :DIM]
    w = rw[:, DIM:]
    rel_re = jnp.cos(r)
    rel_im = jnp.sin(r)

    def rowsum(x):
        return jnp.sum(x, axis=-1, keepdims=True)

    def hyper(x):
        return x - rowsum(w * x) * w

    ph_re = hyper(h_re)
    ph_im = hyper(h_im)
    pt_re = hyper(t_re)
    pt_im = hyper(t_im)
    s_re = ph_re * rel_re - ph_im * rel_im - pt_re
    s_im = ph_re * rel_im + ph_im * rel_re - pt_im
    score = jnp.sqrt(s_re * s_re + s_im * s_im)
    o_ref[...] = rowsum(score) - GAMMA


def _tc_score(h, t, r):
    blk = 2048
    spec = pl.BlockSpec((blk, 2 * DIM), lambda i: (i, 0))
    return pl.pallas_call(
        _tc_body,
        grid=(B // blk,),
        in_specs=[spec] * 3,
        out_specs=pl.BlockSpec((blk, 1), lambda i: (i, 0)),
        out_shape=jax.ShapeDtypeStruct((B, 1), jnp.float32),
    )(h, t, r)


def kernel(pos_sample, ent_embd, ent_embd_im, rel_embd, wr):
    h_idx = pos_sample[:, 0]
    r_idx = pos_sample[:, 1]
    t_idx = pos_sample[:, 2]
    entcat = _tc_concat(ent_embd.T, ent_embd_im.T)
    h, t = _sc_gather2(entcat, h_idx, t_idx)
    relcat = _tc_concat(rel_embd.T, wr.T)
    r = _sc_gather1(relcat, r_idx)
    return _tc_score(h, t, r)


# speedup vs baseline: 1.4305x; 1.4305x over previous
import functools

import jax
import jax.numpy as jnp
from jax import lax
from jax.experimental import pallas as pl
from jax.experimental.pallas import tpu as pltpu
from jax.experimental.pallas import tpu_sc as plsc

ENT_NUM = 100000
DIM = 64
B = 16384
GAMMA = 12.0

_NC = 2
_NS = 16
_NW = _NC * _NS
_BPW = B // _NW
_CHUNK = 256
_NCHUNK = _BPW // _CHUNK


def _concat_body(a_ref, b_ref, o_ref):
    ab = jnp.concatenate([a_ref[...], b_ref[...]], axis=0)
    eye = jnp.eye(2 * DIM, dtype=jnp.float32)
    o_ref[...] = jax.lax.dot_general(
        ab, eye, (((0,), (0,)), ((), ())),
        preferred_element_type=jnp.float32)


def _tc_concat(a, b):
    blk = 4096
    spec = pl.BlockSpec((DIM, blk), lambda i: (0, i))
    return pl.pallas_call(
        _concat_body,
        grid=(pl.cdiv(ENT_NUM, blk),),
        in_specs=[spec, spec],
        out_specs=pl.BlockSpec((blk, 2 * DIM), lambda i: (i, 0)),
        out_shape=jax.ShapeDtypeStruct((ENT_NUM, 2 * DIM), jnp.float32),
    )(a, b)


def _sc_gather2(table, idx0, idx1):
    mesh = plsc.VectorSubcoreMesh(core_axis_name="c", subcore_axis_name="s")
    out_t = [jax.ShapeDtypeStruct((B, 2 * DIM), jnp.float32) for _ in range(2)]

    @functools.partial(
        pl.kernel,
        mesh=mesh,
        out_type=out_t,
        scratch_types=[
            pltpu.VMEM((_BPW,), jnp.int32),
            pltpu.VMEM((_BPW,), jnp.int32),
            pltpu.VMEM((_CHUNK, 2 * DIM), jnp.float32),
            pltpu.VMEM((_CHUNK, 2 * DIM), jnp.float32),
            pltpu.SemaphoreType.DMA,
        ],
    )
    def k(t_hbm, i0_hbm, i1_hbm, o0, o1, i0_v, i1_v, b0, b1, sem):
        wid = lax.axis_index("s") * _NC + lax.axis_index("c")
        base = wid * _BPW
        pltpu.sync_copy(i0_hbm.at[pl.ds(base, _BPW)], i0_v)
        pltpu.sync_copy(i1_hbm.at[pl.ds(base, _BPW)], i1_v)

        def body(c, _):
            off = c * _CHUNK
            cp0 = pltpu.async_copy(t_hbm.at[i0_v.at[pl.ds(off, _CHUNK)]], b0, sem)
            cp1 = pltpu.async_copy(t_hbm.at[i1_v.at[pl.ds(off, _CHUNK)]], b1, sem)
            cp0.wait()
            cp1.wait()
            dst = pl.ds(base + off, _CHUNK)
            pltpu.sync_copy(b0, o0.at[dst])
            pltpu.sync_copy(b1, o1.at[dst])

        lax.fori_loop(0, _NCHUNK, body, None, unroll=False)

    return k(table, idx0, idx1)


def _sc_gather1(table, idx0):
    mesh = plsc.VectorSubcoreMesh(core_axis_name="c", subcore_axis_name="s")
    out_t = jax.ShapeDtypeStruct((B, 2 * DIM), jnp.float32)

    @functools.partial(
        pl.kernel,
        mesh=mesh,
        out_type=out_t,
        scratch_types=[
            pltpu.VMEM((_BPW,), jnp.int32),
            pltpu.VMEM((_CHUNK, 2 * DIM), jnp.float32),
            pltpu.SemaphoreType.DMA,
        ],
    )
    def k(t_hbm, i0_hbm, o0, i0_v, b0, sem):
        wid = lax.axis_index("s") * _NC + lax.axis_index("c")
        base = wid * _BPW
        pltpu.sync_copy(i0_hbm.at[pl.ds(base, _BPW)], i0_v)

        def body(c, _):
            off = c * _CHUNK
            cp0 = pltpu.async_copy(t_hbm.at[i0_v.at[pl.ds(off, _CHUNK)]], b0, sem)
            cp0.wait()
            pltpu.sync_copy(b0, o0.at[pl.ds(base + off, _CHUNK)])

        lax.fori_loop(0, _NCHUNK, body, None, unroll=False)

    return k(table, idx0)


def _tc_body(h_ref, t_ref, r_ref, o_ref):
    h = h_ref[...]
    t = t_ref[...]
    rw = r_ref[...]
    h_re = h[:, :DIM]
    h_im = h[:, DIM:]
    t_re = t[:, :DIM]
    t_im = t[:, DIM:]
    r = rw[:, :DIM]
    w = rw[:, DIM:]
    rel_re = jnp.cos(r)
    rel_im = jnp.sin(r)

    def rowsum(x):
        return jnp.sum(x, axis=-1, keepdims=True)

    def hyper(x):
        return x - rowsum(w * x) * w

    ph_re = hyper(h_re)
    ph_im = hyper(h_im)
    pt_re = hyper(t_re)
    pt_im = hyper(t_im)
    s_re = ph_re * rel_re - ph_im * rel_im - pt_re
    s_im = ph_re * rel_im + ph_im * rel_re - pt_im
    score = jnp.sqrt(s_re * s_re + s_im * s_im)
    o_ref[...] = rowsum(score) - GAMMA


def _tc_score(h, t, r):
    blk = 2048
    spec = pl.BlockSpec((blk, 2 * DIM), lambda i: (i, 0))
    return pl.pallas_call(
        _tc_body,
        grid=(B // blk,),
        in_specs=[spec] * 3,
        out_specs=pl.BlockSpec((blk, 1), lambda i: (i, 0)),
        out_shape=jax.ShapeDtypeStruct((B, 1), jnp.float32),
    )(h, t, r)


def kernel(pos_sample, ent_embd, ent_embd_im, rel_embd, wr):
    h_idx = pos_sample[:, 0]
    r_idx = pos_sample[:, 1]
    t_idx = pos_sample[:, 2]
    entcat = _tc_concat(ent_embd.T, ent_embd_im.T)
    h, t = _sc_gather2(entcat, h_idx, t_idx)
    relcat = _tc_concat(rel_embd.T, wr.T)
    r = _sc_gather1(relcat, r_idx)
    return _tc_score(h, t, r)
